# baseline (speedup 1.0000x reference)
import functools
import jax
import jax.numpy as jnp
from jax import lax
from jax.experimental import pallas as pl
from jax.experimental.pallas import tpu as pltpu
from jax.experimental.pallas import tpu_sc as plsc

N = 10000
D = 128
E = 320000

NC = 2
NS = 16
NW = NC * NS

NR = 10368
NP = 10240
NB = 10368
NT = 10112
NI = 10048
DH = 64
EPS = 20224
CH = 316
CW = 64
SENT = 10000
JSENT = 10016
PTRASH = 10032

_mesh = plsc.VectorSubcoreMesh(core_axis_name="c", subcore_axis_name="s")
_f32 = jnp.float32
_i32 = jnp.int32


@functools.partial(
    pl.kernel,
    out_type=[
        jax.ShapeDtypeStruct((NC, NT, DH), _f32),
        jax.ShapeDtypeStruct((NS, NB, 8), _i32),
        jax.ShapeDtypeStruct((NS, NB, 8), _i32),
    ],
    mesh=_mesh,
    compiler_params=pltpu.CompilerParams(needs_layout_passes=False, use_tc_tiling_on_sc=False),
    scratch_types=[
        pltpu.VMEM((1, 2 * CW), _i32),
        pltpu.VMEM((1, 2 * CW), _i32),
        pltpu.VMEM((CW, DH), _f32),
        pltpu.VMEM((CW, DH), _f32),
        pltpu.VMEM((CW, 8), _i32),
        pltpu.VMEM_SHARED((NT, DH), _f32),
        pltpu.SemaphoreType.DMA,
        pltpu.SemaphoreType.DMA,
        pltpu.SemaphoreType.DMA,
        pltpu.SemaphoreType.DMA,
        pltpu.SemaphoreType.DMA,
        pltpu.SemaphoreType.DMA,
        pltpu.SemaphoreType.DMA,
        pltpu.SemaphoreType.DMA,
    ],
)
def _sc_edge_agg(hlo_hbm, hhi_hbm, sd_hbm, ones_hbm, zt_hbm,
                 zp_hbm,
                 t_out, ps_out, pd_out,
                 ib0, ib1, rows0, rows1, ones8, t_sp,
                 semi0, semi1, semg0, semg1,
                 semsc0, semsc1, semp0, semp1):
    c = lax.axis_index("c")
    s = lax.axis_index("s")
    rpt = NT // NS

    pltpu.sync_copy(ones_hbm, ones8)
    pltpu.sync_copy(zt_hbm, t_sp.at[pl.ds(s * rpt, rpt)])

    @pl.when(c == 0)
    def _():
        pltpu.sync_copy(zp_hbm, ps_out.at[s])
        pltpu.sync_copy(zp_hbm, pd_out.at[s])

    plsc.subcore_barrier()

    ib = (ib0, ib1)
    rows = (rows0, rows1)
    semi = (semi0, semi1)
    semg = (semg0, semg1)
    semsc = (semsc0, semsc1)
    semp = (semp0, semp1)

    def fire_idx(q, p):
        pltpu.async_copy(sd_hbm.at[s, pl.ds(q, 1)], ib[p], semi[p])

    def wait_idx(p):
        pltpu.make_async_copy(sd_hbm.at[s, pl.ds(0, 1)], ib[p],
                              semi[p]).wait()

    def wait_rows(sem, p):
        pltpu.make_async_copy(hlo_hbm.at[pl.ds(0, CW)], rows[p], sem).wait()

    def wait_pres(p):
        pltpu.make_async_copy(zp_hbm.at[pl.ds(0, CW)], ones8, semp[p]).wait()
        pltpu.make_async_copy(zp_hbm.at[pl.ds(0, CW)], ones8, semp[p]).wait()

    fire_idx(0, 0)
    fire_idx(1, 1)

    def halfstep(i, p):
        wait_idx(p)

        isrc = ib[p].at[0, pl.ds(0, CW)]
        idst = ib[p].at[0, pl.ds(CW, CW)]

        @pl.when(c == 0)
        def _():
            pltpu.async_copy(hlo_hbm.at[idst], rows[p], semg[p])

        @pl.when(c == 1)
        def _():
            pltpu.async_copy(hhi_hbm.at[idst], rows[p], semg[p])

        wait_rows(semg[p], p)
        pltpu.async_copy(rows[p], t_sp.at[isrc], semsc[p], add=True)

        @pl.when(c == 0)
        def _():
            pltpu.async_copy(ones8, ps_out.at[s].at[isrc], semp[p])
            pltpu.async_copy(ones8, pd_out.at[s].at[idst], semp[p])

    def refill(i, p):
        wait_rows(semsc[p], p)

        @pl.when(c == 0)
        def _():
            wait_pres(p)

        @pl.when(i < CH // 2 - 1)
        def _():
            fire_idx(2 * i + 2 + p, p)

    def body(i, carry):
        halfstep(i, 0)
        halfstep(i, 1)
        refill(i, 0)
        refill(i, 1)
        return carry

    lax.fori_loop(0, CH // 2, body, 0)

    plsc.subcore_barrier()

    pltpu.sync_copy(t_sp.at[pl.ds(s * rpt, rpt)],
                    t_out.at[c, pl.ds(s * rpt, rpt)])


def _tc_rank_body(t2_ref, bs_ref, bd_ref, text_ref, jf_ref, pos_ref, pd_ref):
    cs_t = jnp.sum(bs_ref[...], axis=0)
    cd_t = jnp.sum(bd_ref[...], axis=0)
    nrow = NB // 128
    r = lax.broadcasted_iota(_i32, (nrow, 128), 0)
    cl = lax.broadcasted_iota(_i32, (nrow, 128), 1)
    v = r * 128 + cl
    valid = v < N
    ps = (cs_t > 0) & valid
    pd = (cd_t > 0) & valid
    psf = ps.astype(_f32)
    pdf = pd.astype(_f32)
    u = (lax.broadcasted_iota(_i32, (128, 128), 0)
         <= lax.broadcasted_iota(_i32, (128, 128), 1)).astype(_f32)
    incl_s = jnp.dot(psf, u, preferred_element_type=_f32)
    incl_d = jnp.dot(pdf, u, preferred_element_type=_f32)
    sl = (lax.broadcasted_iota(_i32, (nrow, nrow), 0)
          > lax.broadcasted_iota(_i32, (nrow, nrow), 1)).astype(_f32)
    off_s = jnp.dot(sl, incl_s[:, 127:128], preferred_element_type=_f32)
    off_d = jnp.dot(sl, incl_d[:, 127:128], preferred_element_type=_f32)
    rs = off_s + incl_s - psf
    rd = off_d + incl_d - pdf
    n_src = jnp.sum(psf)
    jf = jnp.where(pd & (rd < n_src), rd, float(JSENT)).astype(_i32)
    pos = jnp.where(ps, rs, float(PTRASH)).astype(_i32)
    jf_ref[...] = jf[: NP // 128]
    pos_ref[...] = pos[: NP // 128]
    pd_ref[...] = pd[: NP // 128].astype(_i32)
    rowmask = (lax.broadcasted_iota(_i32, (NT, D), 0) < N).astype(_f32)
    tt = jnp.concatenate([t2_ref[0], t2_ref[1]], axis=1)
    text_ref[pl.ds(0, NT), :] = tt * rowmask
    text_ref[pl.ds(NT, NR - NT), :] = jnp.zeros((NR - NT, D), _f32)


_tc_rank = pl.pallas_call(
    _tc_rank_body,
    out_shape=[
        jax.ShapeDtypeStruct((NR, D), _f32),
        jax.ShapeDtypeStruct((NP // 128, 128), _i32),
        jax.ShapeDtypeStruct((NP // 128, 128), _i32),
        jax.ShapeDtypeStruct((NP // 128, 128), _i32),
    ],
)


@functools.partial(
    pl.kernel,
    out_type=jax.ShapeDtypeStruct((NP, D), _f32),
    mesh=_mesh,
    compiler_params=pltpu.CompilerParams(needs_layout_passes=False, use_tc_tiling_on_sc=False),
    scratch_types=[
        pltpu.VMEM((NI,), _i32),
        pltpu.VMEM((512,), _i32),
        pltpu.VMEM((320,), _i32),
        pltpu.VMEM((320,), _i32),
        pltpu.VMEM((32, D), _f32),
    ],
)
def _sc_gather_ranks(text_hbm, pos_hbm, jf_hbm, zi_hbm,
                     g_out,
                     inv, posv, jfv, idx2, grows):
    c = lax.axis_index("c")
    s = lax.axis_index("s")
    wid = c * NS + s
    iota = lax.iota(_i32, 16)

    pltpu.sync_copy(zi_hbm, inv)

    for k in range(NP // 512):
        pltpu.sync_copy(pos_hbm.at[pl.ds(k * 512, 512)], posv)

        def inv_body(t, carry, k=k):
            pv = posv[pl.ds(t * 16, 16)]
            plsc.store_scatter(inv, [pv], iota + (k * 512 + t * 16))
            return carry
        lax.fori_loop(0, 32, inv_body, 0)
    plsc.store_scatter(inv, [iota + JSENT], jnp.full((16,), SENT, _i32))

    base = wid * 320
    pltpu.sync_copy(jf_hbm.at[pl.ds(base, 320)], jfv)
    for k in range(20):
        jv = jfv[pl.ds(k * 16, 16)]
        uv = plsc.load_gather(inv, [jv])
        idx2[pl.ds(k * 16, 16)] = uv
    for t in range(10):
        pltpu.sync_copy(text_hbm.at[idx2.at[pl.ds(t * 32, 32)]], grows)
        pltpu.sync_copy(grows, g_out.at[pl.ds(base + t * 32, 32)])


def _dense_update(g_ref, h_ref, x_ref, pdc_ref, wl_ref, w0_ref, w1_ref,
                  bl_ref, b0_ref, b1_ref, lng_ref, lnb_ref):
    dims = (((1,), (1,)), ((), ()))
    hd = lax.dot_general(g_ref[...], wl_ref[...], dims,
                         preferred_element_type=_f32)
    hd = hd + lax.dot_general(h_ref[...], w0_ref[...], dims,
                              preferred_element_type=_f32)
    hd = hd + lax.dot_general(x_ref[...], w1_ref[...], dims,
                              preferred_element_type=_f32)
    hd = hd + (bl_ref[...] + b0_ref[...] + b1_ref[...])
    hd = jnp.maximum(hd, 0.0)
    mu = jnp.mean(hd, axis=-1, keepdims=True)
    var = jnp.mean((hd - mu) * (hd - mu), axis=-1, keepdims=True)
    hd = (hd - mu) * lax.rsqrt(var + 1e-5) * lng_ref[...] + lnb_ref[...]
    h = h_ref[...]
    return h + pdc_ref[...] * (hd - h)


def _tc_update_body(g_ref, h_ref, x_ref, pdc_ref, wl_ref, w0_ref, w1_ref,
                    bl_ref, b0_ref, b1_ref, lng_ref, lnb_ref, hext_ref):
    hn = _dense_update(g_ref, h_ref, x_ref, pdc_ref, wl_ref, w0_ref, w1_ref,
                       bl_ref, b0_ref, b1_ref, lng_ref, lnb_ref)
    hext_ref[pl.ds(0, N), :] = hn
    hext_ref[pl.ds(N, NR - N), :] = jnp.zeros((NR - N, D), _f32)


def _tc_final_body(h_ref, ow_ref, ob_ref, out_ref):
    dims = (((1,), (1,)), ((), ()))
    out_ref[...] = lax.dot_general(h_ref[...], ow_ref[...], dims,
                                   preferred_element_type=_f32) + ob_ref[...]


_tc_update = pl.pallas_call(
    _tc_update_body, out_shape=jax.ShapeDtypeStruct((NR, D), _f32))
_tc_final = pl.pallas_call(
    _tc_final_body, out_shape=jax.ShapeDtypeStruct((N, 128), _f32))


def _prep_edges(ei):
    pad = NS * EPS - E
    src = jnp.concatenate([ei[0], jnp.full((pad,), SENT, _i32)])
    dst = jnp.concatenate([ei[1], jnp.full((pad,), SENT, _i32)])
    return jnp.concatenate([src.reshape(NS, CH, CW), dst.reshape(NS, CH, CW)],
                           axis=2)


@jax.jit
def kernel(x, edge_index0, edge_index1, wl0_W, wl0_b, w00_W, w00_b, w10_W,
           w10_b, ln0_g, ln0_b, wl1_W, wl1_b, w01_W, w01_b, w11_W, w11_b,
           ln1_g, ln1_b, out_W, out_b):
    ones8 = jnp.ones((CW, 8), _i32)
    zt = jnp.zeros((NT // NS, DH), _f32)
    zp = jnp.zeros((NB, 8), _i32)
    zi = jnp.zeros((NI,), _i32)
    row = lambda b: b.reshape(1, D)

    e1 = _prep_edges(edge_index1)
    e0 = _prep_edges(edge_index0)
    stk = lambda a, b: jnp.stack([a, b])
    xs = (
        stk(e1, e0),
        stk(wl1_W, wl0_W), stk(row(wl1_b), row(wl0_b)),
        stk(w01_W, w00_W), stk(row(w01_b), row(w00_b)),
        stk(w11_W, w10_W), stk(row(w11_b), row(w10_b)),
        stk(row(ln1_g), row(ln0_g)), stk(row(ln1_b), row(ln0_b)),
    )

    def conv_step(h_ext, xv):
        (sd2d, wl, bl, w0, b0, w1, b1, lng, lnb) = xv
        hlo = h_ext[:, :DH]
        hhi = h_ext[:, DH:]
        t2, bs, bd = _sc_edge_agg(hlo, hhi, sd2d, ones8, zt, zp)
        bs1 = bs[:, :, 0].reshape(NS, NB // 128, 128)
        bd1 = bd[:, :, 0].reshape(NS, NB // 128, 128)
        text, jf, pos, pdm = _tc_rank(t2, bs1, bd1)
        g = _sc_gather_ranks(text, pos.reshape(NP), jf.reshape(NP), zi)
        pdc = pdm.reshape(NP)[:N, None].astype(_f32)
        h_new = _tc_update(g[:N], h_ext[:N], x, pdc, wl, w0, w1,
                           bl, b0, b1, lng, lnb)
        return h_new, None

    h_ext = jnp.pad(x, ((0, NR - N), (0, 0)))
    h_ext, _ = lax.scan(conv_step, h_ext, xs)
    return _tc_final(h_ext[:N], out_W, row(out_b))


# speedup vs baseline: 1.0107x; 1.0107x over previous
import functools
import jax
import jax.numpy as jnp
from jax import lax
from jax.experimental import pallas as pl
from jax.experimental.pallas import tpu as pltpu
from jax.experimental.pallas import tpu_sc as plsc

N = 10000
D = 128
E = 320000

NC = 2
NS = 16
NW = NC * NS

NR = 10368
NP = 10240
NB = 10368
NT = 10112
NI = 10048
DH = 64
EPS = 20224
CH = 316
CW = 64
SENT = 10000
JSENT = 10016
PTRASH = 10032

_mesh = plsc.VectorSubcoreMesh(core_axis_name="c", subcore_axis_name="s")
_f32 = jnp.float32
_i32 = jnp.int32


@functools.partial(
    pl.kernel,
    out_type=[
        jax.ShapeDtypeStruct((NC, NT, DH), _f32),
        jax.ShapeDtypeStruct((NS, NB, 8), _i32),
        jax.ShapeDtypeStruct((NS, NB, 8), _i32),
    ],
    mesh=_mesh,
    compiler_params=pltpu.CompilerParams(needs_layout_passes=False, use_tc_tiling_on_sc=False),
    scratch_types=[
        pltpu.VMEM((1, 2 * CW), _i32),
        pltpu.VMEM((1, 2 * CW), _i32),
        pltpu.VMEM((CW, DH), _f32),
        pltpu.VMEM((CW, DH), _f32),
        pltpu.VMEM((CW, 8), _i32),
        pltpu.VMEM_SHARED((NT, DH), _f32),
        pltpu.SemaphoreType.DMA,
        pltpu.SemaphoreType.DMA,
        pltpu.SemaphoreType.DMA,
        pltpu.SemaphoreType.DMA,
        pltpu.SemaphoreType.DMA,
        pltpu.SemaphoreType.DMA,
        pltpu.SemaphoreType.DMA,
        pltpu.SemaphoreType.DMA,
    ],
)
def _sc_edge_agg(hlo_hbm, hhi_hbm, sd_hbm, ones_hbm, zt_hbm,
                 zp_hbm,
                 t_out, ps_out, pd_out,
                 ib0, ib1, rows0, rows1, ones8, t_sp,
                 semi0, semi1, semg0, semg1,
                 semsc0, semsc1, semp0, semp1):
    c = lax.axis_index("c")
    s = lax.axis_index("s")
    rpt = NT // NS

    pltpu.sync_copy(ones_hbm, ones8)
    pltpu.sync_copy(zt_hbm, t_sp.at[pl.ds(s * rpt, rpt)])

    @pl.when(c == 0)
    def _():
        pltpu.sync_copy(zp_hbm, ps_out.at[s])
        pltpu.sync_copy(zp_hbm, pd_out.at[s])

    plsc.subcore_barrier()

    ib = (ib0, ib1)
    rows = (rows0, rows1)
    semi = (semi0, semi1)
    semg = (semg0, semg1)
    semsc = (semsc0, semsc1)
    semp = (semp0, semp1)

    def fire_idx(q, p):
        pltpu.async_copy(sd_hbm.at[s, pl.ds(q, 1)], ib[p], semi[p])

    def wait_idx(p):
        pltpu.make_async_copy(sd_hbm.at[s, pl.ds(0, 1)], ib[p],
                              semi[p]).wait()

    def wait_rows(sem, p):
        pltpu.make_async_copy(hlo_hbm.at[pl.ds(0, CW // 2)],
                              rows[p].at[pl.ds(0, CW // 2)], sem).wait()
        pltpu.make_async_copy(hlo_hbm.at[pl.ds(0, CW // 2)],
                              rows[p].at[pl.ds(0, CW // 2)], sem).wait()

    def wait_pres(p):
        pltpu.make_async_copy(zp_hbm.at[pl.ds(0, CW)], ones8, semp[p]).wait()
        pltpu.make_async_copy(zp_hbm.at[pl.ds(0, CW)], ones8, semp[p]).wait()

    fire_idx(0, 0)
    fire_idx(1, 1)

    def halfstep(i, p):
        wait_idx(p)

        isrc = ib[p].at[0, pl.ds(0, CW)]
        idst = ib[p].at[0, pl.ds(CW, CW)]
        hw = CW // 2
        idst_a = ib[p].at[0, pl.ds(CW, hw)]
        idst_b = ib[p].at[0, pl.ds(CW + hw, hw)]
        isrc_a = ib[p].at[0, pl.ds(0, hw)]
        isrc_b = ib[p].at[0, pl.ds(hw, hw)]
        rlo = rows[p].at[pl.ds(0, hw)]
        rhi = rows[p].at[pl.ds(hw, hw)]

        @pl.when(c == 0)
        def _():
            pltpu.async_copy(hlo_hbm.at[idst_a], rlo, semg[p])
            pltpu.async_copy(hlo_hbm.at[idst_b], rhi, semg[p])

        @pl.when(c == 1)
        def _():
            pltpu.async_copy(hhi_hbm.at[idst_a], rlo, semg[p])
            pltpu.async_copy(hhi_hbm.at[idst_b], rhi, semg[p])

        wait_rows(semg[p], p)
        pltpu.async_copy(rlo, t_sp.at[isrc_a], semsc[p], add=True)
        pltpu.async_copy(rhi, t_sp.at[isrc_b], semsc[p], add=True)

        @pl.when(c == 0)
        def _():
            pltpu.async_copy(ones8, ps_out.at[s].at[isrc], semp[p])
            pltpu.async_copy(ones8, pd_out.at[s].at[idst], semp[p])

    def refill(i, p):
        wait_rows(semsc[p], p)

        @pl.when(c == 0)
        def _():
            wait_pres(p)

        @pl.when(i < CH // 2 - 1)
        def _():
            fire_idx(2 * i + 2 + p, p)

    def body(i, carry):
        halfstep(i, 0)
        halfstep(i, 1)
        refill(i, 0)
        refill(i, 1)
        return carry

    lax.fori_loop(0, CH // 2, body, 0)

    plsc.subcore_barrier()

    pltpu.sync_copy(t_sp.at[pl.ds(s * rpt, rpt)],
                    t_out.at[c, pl.ds(s * rpt, rpt)])


def _tc_rank_body(t2_ref, bs_ref, bd_ref, text_ref, jf_ref, pos_ref, pd_ref):
    cs_t = jnp.sum(bs_ref[...], axis=0)
    cd_t = jnp.sum(bd_ref[...], axis=0)
    nrow = NB // 128
    r = lax.broadcasted_iota(_i32, (nrow, 128), 0)
    cl = lax.broadcasted_iota(_i32, (nrow, 128), 1)
    v = r * 128 + cl
    valid = v < N
    ps = (cs_t > 0) & valid
    pd = (cd_t > 0) & valid
    psf = ps.astype(_f32)
    pdf = pd.astype(_f32)
    u = (lax.broadcasted_iota(_i32, (128, 128), 0)
         <= lax.broadcasted_iota(_i32, (128, 128), 1)).astype(_f32)
    incl_s = jnp.dot(psf, u, preferred_element_type=_f32)
    incl_d = jnp.dot(pdf, u, preferred_element_type=_f32)
    sl = (lax.broadcasted_iota(_i32, (nrow, nrow), 0)
          > lax.broadcasted_iota(_i32, (nrow, nrow), 1)).astype(_f32)
    off_s = jnp.dot(sl, incl_s[:, 127:128], preferred_element_type=_f32)
    off_d = jnp.dot(sl, incl_d[:, 127:128], preferred_element_type=_f32)
    rs = off_s + incl_s - psf
    rd = off_d + incl_d - pdf
    n_src = jnp.sum(psf)
    jf = jnp.where(pd & (rd < n_src), rd, float(JSENT)).astype(_i32)
    pos = jnp.where(ps, rs, float(PTRASH)).astype(_i32)
    jf_ref[...] = jf[: NP // 128]
    pos_ref[...] = pos[: NP // 128]
    pd_ref[...] = pd[: NP // 128].astype(_i32)
    rowmask = (lax.broadcasted_iota(_i32, (NT, D), 0) < N).astype(_f32)
    tt = jnp.concatenate([t2_ref[0], t2_ref[1]], axis=1)
    text_ref[pl.ds(0, NT), :] = tt * rowmask
    text_ref[pl.ds(NT, NR - NT), :] = jnp.zeros((NR - NT, D), _f32)


_tc_rank = pl.pallas_call(
    _tc_rank_body,
    out_shape=[
        jax.ShapeDtypeStruct((NR, D), _f32),
        jax.ShapeDtypeStruct((NP // 128, 128), _i32),
        jax.ShapeDtypeStruct((NP // 128, 128), _i32),
        jax.ShapeDtypeStruct((NP // 128, 128), _i32),
    ],
)


@functools.partial(
    pl.kernel,
    out_type=jax.ShapeDtypeStruct((NP, D), _f32),
    mesh=_mesh,
    compiler_params=pltpu.CompilerParams(needs_layout_passes=False, use_tc_tiling_on_sc=False),
    scratch_types=[
        pltpu.VMEM((NI,), _i32),
        pltpu.VMEM((512,), _i32),
        pltpu.VMEM((320,), _i32),
        pltpu.VMEM((320,), _i32),
        pltpu.VMEM((32, D), _f32),
    ],
)
def _sc_gather_ranks(text_hbm, pos_hbm, jf_hbm, zi_hbm,
                     g_out,
                     inv, posv, jfv, idx2, grows):
    c = lax.axis_index("c")
    s = lax.axis_index("s")
    wid = c * NS + s
    iota = lax.iota(_i32, 16)

    pltpu.sync_copy(zi_hbm, inv)

    for k in range(NP // 512):
        pltpu.sync_copy(pos_hbm.at[pl.ds(k * 512, 512)], posv)

        def inv_body(t, carry, k=k):
            pv = posv[pl.ds(t * 16, 16)]
            plsc.store_scatter(inv, [pv], iota + (k * 512 + t * 16))
            return carry
        lax.fori_loop(0, 32, inv_body, 0)
    plsc.store_scatter(inv, [iota + JSENT], jnp.full((16,), SENT, _i32))

    base = wid * 320
    pltpu.sync_copy(jf_hbm.at[pl.ds(base, 320)], jfv)
    for k in range(20):
        jv = jfv[pl.ds(k * 16, 16)]
        uv = plsc.load_gather(inv, [jv])
        idx2[pl.ds(k * 16, 16)] = uv
    for t in range(10):
        pltpu.sync_copy(text_hbm.at[idx2.at[pl.ds(t * 32, 32)]], grows)
        pltpu.sync_copy(grows, g_out.at[pl.ds(base + t * 32, 32)])


def _dense_update(g_ref, h_ref, x_ref, pdc_ref, wl_ref, w0_ref, w1_ref,
                  bl_ref, b0_ref, b1_ref, lng_ref, lnb_ref):
    dims = (((1,), (1,)), ((), ()))
    hd = lax.dot_general(g_ref[...], wl_ref[...], dims,
                         preferred_element_type=_f32)
    hd = hd + lax.dot_general(h_ref[...], w0_ref[...], dims,
                              preferred_element_type=_f32)
    hd = hd + lax.dot_general(x_ref[...], w1_ref[...], dims,
                              preferred_element_type=_f32)
    hd = hd + (bl_ref[...] + b0_ref[...] + b1_ref[...])
    hd = jnp.maximum(hd, 0.0)
    mu = jnp.mean(hd, axis=-1, keepdims=True)
    var = jnp.mean((hd - mu) * (hd - mu), axis=-1, keepdims=True)
    hd = (hd - mu) * lax.rsqrt(var + 1e-5) * lng_ref[...] + lnb_ref[...]
    h = h_ref[...]
    return h + pdc_ref[...] * (hd - h)


def _tc_update_body(g_ref, h_ref, x_ref, pdc_ref, wl_ref, w0_ref, w1_ref,
                    bl_ref, b0_ref, b1_ref, lng_ref, lnb_ref, hext_ref):
    hn = _dense_update(g_ref, h_ref, x_ref, pdc_ref, wl_ref, w0_ref, w1_ref,
                       bl_ref, b0_ref, b1_ref, lng_ref, lnb_ref)
    hext_ref[pl.ds(0, N), :] = hn
    hext_ref[pl.ds(N, NR - N), :] = jnp.zeros((NR - N, D), _f32)


def _tc_final_body(h_ref, ow_ref, ob_ref, out_ref):
    dims = (((1,), (1,)), ((), ()))
    out_ref[...] = lax.dot_general(h_ref[...], ow_ref[...], dims,
                                   preferred_element_type=_f32) + ob_ref[...]


_tc_update = pl.pallas_call(
    _tc_update_body, out_shape=jax.ShapeDtypeStruct((NR, D), _f32))
_tc_final = pl.pallas_call(
    _tc_final_body, out_shape=jax.ShapeDtypeStruct((N, 128), _f32))


def _prep_edges(ei):
    pad = NS * EPS - E
    src = jnp.concatenate([ei[0], jnp.full((pad,), SENT, _i32)])
    dst = jnp.concatenate([ei[1], jnp.full((pad,), SENT, _i32)])
    return jnp.concatenate([src.reshape(NS, CH, CW), dst.reshape(NS, CH, CW)],
                           axis=2)


@jax.jit
def kernel(x, edge_index0, edge_index1, wl0_W, wl0_b, w00_W, w00_b, w10_W,
           w10_b, ln0_g, ln0_b, wl1_W, wl1_b, w01_W, w01_b, w11_W, w11_b,
           ln1_g, ln1_b, out_W, out_b):
    ones8 = jnp.ones((CW, 8), _i32)
    zt = jnp.zeros((NT // NS, DH), _f32)
    zp = jnp.zeros((NB, 8), _i32)
    zi = jnp.zeros((NI,), _i32)
    row = lambda b: b.reshape(1, D)

    e1 = _prep_edges(edge_index1)
    e0 = _prep_edges(edge_index0)
    stk = lambda a, b: jnp.stack([a, b])
    xs = (
        stk(e1, e0),
        stk(wl1_W, wl0_W), stk(row(wl1_b), row(wl0_b)),
        stk(w01_W, w00_W), stk(row(w01_b), row(w00_b)),
        stk(w11_W, w10_W), stk(row(w11_b), row(w10_b)),
        stk(row(ln1_g), row(ln0_g)), stk(row(ln1_b), row(ln0_b)),
    )

    def conv_step(h_ext, xv):
        (sd2d, wl, bl, w0, b0, w1, b1, lng, lnb) = xv
        hlo = h_ext[:, :DH]
        hhi = h_ext[:, DH:]
        t2, bs, bd = _sc_edge_agg(hlo, hhi, sd2d, ones8, zt, zp)
        bs1 = bs[:, :, 0].reshape(NS, NB // 128, 128)
        bd1 = bd[:, :, 0].reshape(NS, NB // 128, 128)
        text, jf, pos, pdm = _tc_rank(t2, bs1, bd1)
        g = _sc_gather_ranks(text, pos.reshape(NP), jf.reshape(NP), zi)
        pdc = pdm.reshape(NP)[:N, None].astype(_f32)
        h_new = _tc_update(g[:N], h_ext[:N], x, pdc, wl, w0, w1,
                           bl, b0, b1, lng, lnb)
        return h_new, None

    h_ext = jnp.pad(x, ((0, NR - N), (0, 0)))
    h_ext, _ = lax.scan(conv_step, h_ext, xs)
    return _tc_final(h_ext[:N], out_W, row(out_b))


# speedup vs baseline: 1.0358x; 1.0248x over previous
import functools
import jax
import jax.numpy as jnp
from jax import lax
from jax.experimental import pallas as pl
from jax.experimental.pallas import tpu as pltpu
from jax.experimental.pallas import tpu_sc as plsc

N = 10000
D = 128
E = 320000

NC = 2
NS = 16
NW = NC * NS

NR = 10368
NP = 10240
NB = 10368
NT = 10112
NI = 10048
DH = 64
EPS = 20224
CH = 316
CW = 64
SENT = 10000
JSENT = 10016
PTRASH = 10032

_mesh = plsc.VectorSubcoreMesh(core_axis_name="c", subcore_axis_name="s")
_f32 = jnp.float32
_i32 = jnp.int32


@functools.partial(
    pl.kernel,
    out_type=[
        jax.ShapeDtypeStruct((NC, NT, DH), _f32),
        jax.ShapeDtypeStruct((NS, NB, 8), _i32),
        jax.ShapeDtypeStruct((NS, NB, 8), _i32),
    ],
    mesh=_mesh,
    compiler_params=pltpu.CompilerParams(needs_layout_passes=False, use_tc_tiling_on_sc=False),
    scratch_types=[
        pltpu.VMEM((1, 2 * CW), _i32),
        pltpu.VMEM((1, 2 * CW), _i32),
        pltpu.VMEM((CW, DH), _f32),
        pltpu.VMEM((CW, DH), _f32),
        pltpu.VMEM((CW, 8), _i32),
        pltpu.VMEM_SHARED((NT, DH), _f32),
        pltpu.SemaphoreType.DMA,
        pltpu.SemaphoreType.DMA,
        pltpu.SemaphoreType.DMA,
        pltpu.SemaphoreType.DMA,
        pltpu.SemaphoreType.DMA,
        pltpu.SemaphoreType.DMA,
        pltpu.SemaphoreType.DMA,
        pltpu.SemaphoreType.DMA,
    ],
)
def _sc_edge_agg(hlo_hbm, hhi_hbm, sd_hbm, ones_hbm, zt_hbm,
                 zp_hbm,
                 t_out, ps_out, pd_out,
                 ib0, ib1, rows0, rows1, ones8, t_sp,
                 semi0, semi1, semg0, semg1,
                 semsc0, semsc1, semp0, semp1):
    c = lax.axis_index("c")
    s = lax.axis_index("s")
    rpt = NT // NS

    pltpu.sync_copy(ones_hbm, ones8)
    pltpu.sync_copy(zt_hbm, t_sp.at[pl.ds(s * rpt, rpt)])

    @pl.when(c == 0)
    def _():
        pltpu.sync_copy(zp_hbm, ps_out.at[s])
        pltpu.sync_copy(zp_hbm, pd_out.at[s])

    plsc.subcore_barrier()

    ib = (ib0, ib1)
    rows = (rows0, rows1)
    semi = (semi0, semi1)
    semg = (semg0, semg1)
    semsc = (semsc0, semsc1)
    semp = (semp0, semp1)

    def fire_idx(q, p):
        pltpu.async_copy(sd_hbm.at[s, pl.ds(q, 1)], ib[p], semi[p])

    def wait_idx(p):
        pltpu.make_async_copy(sd_hbm.at[s, pl.ds(0, 1)], ib[p],
                              semi[p]).wait()

    def wait_rows(sem, p):
        pltpu.make_async_copy(hlo_hbm.at[pl.ds(0, CW // 2)],
                              rows[p].at[pl.ds(0, CW // 2)], sem).wait()
        pltpu.make_async_copy(hlo_hbm.at[pl.ds(0, CW // 2)],
                              rows[p].at[pl.ds(0, CW // 2)], sem).wait()

    def wait_pres(p):
        pltpu.make_async_copy(zp_hbm.at[pl.ds(0, CW)], ones8, semp[p]).wait()
        pltpu.make_async_copy(zp_hbm.at[pl.ds(0, CW)], ones8, semp[p]).wait()

    fire_idx(0, 0)
    fire_idx(1, 1)

    def halfstep(i, p):
        wait_idx(p)

        isrc = ib[p].at[0, pl.ds(0, CW)]
        idst = ib[p].at[0, pl.ds(CW, CW)]
        hw = CW // 2
        idst_a = ib[p].at[0, pl.ds(CW, hw)]
        idst_b = ib[p].at[0, pl.ds(CW + hw, hw)]
        isrc_a = ib[p].at[0, pl.ds(0, hw)]
        isrc_b = ib[p].at[0, pl.ds(hw, hw)]
        rlo = rows[p].at[pl.ds(0, hw)]
        rhi = rows[p].at[pl.ds(hw, hw)]

        @pl.when(c == 0)
        def _():
            pltpu.async_copy(hlo_hbm.at[idst_a], rlo, semg[p])
            pltpu.async_copy(hlo_hbm.at[idst_b], rhi, semg[p])

        @pl.when(c == 1)
        def _():
            pltpu.async_copy(hhi_hbm.at[idst_a], rlo, semg[p])
            pltpu.async_copy(hhi_hbm.at[idst_b], rhi, semg[p])

        wait_rows(semg[p], p)
        pltpu.async_copy(rlo, t_sp.at[isrc_a], semsc[p], add=True)
        pltpu.async_copy(rhi, t_sp.at[isrc_b], semsc[p], add=True)

        @pl.when(c == 0)
        def _():
            pltpu.async_copy(ones8, ps_out.at[s].at[isrc], semp[p])
            pltpu.async_copy(ones8, pd_out.at[s].at[idst], semp[p])

    def refill(i, p):
        wait_rows(semsc[p], p)

        @pl.when(c == 0)
        def _():
            wait_pres(p)

        @pl.when(i < CH // 2 - 1)
        def _():
            fire_idx(2 * i + 2 + p, p)

    def body(i, carry):
        halfstep(i, 0)
        halfstep(i, 1)
        refill(i, 0)
        refill(i, 1)
        return carry

    lax.fori_loop(0, CH // 2, body, 0)

    plsc.subcore_barrier()

    pltpu.sync_copy(t_sp.at[pl.ds(s * rpt, rpt)],
                    t_out.at[c, pl.ds(s * rpt, rpt)])


def _tc_rank_body(t2_ref, bs_ref, bd_ref, text_ref, jf_ref, pos_ref, pd_ref):
    cs_t = jnp.sum(bs_ref[...], axis=0)
    cd_t = jnp.sum(bd_ref[...], axis=0)
    nrow = NB // 128
    r = lax.broadcasted_iota(_i32, (nrow, 128), 0)
    cl = lax.broadcasted_iota(_i32, (nrow, 128), 1)
    v = r * 128 + cl
    valid = v < N
    ps = (cs_t > 0) & valid
    pd = (cd_t > 0) & valid
    psf = ps.astype(_f32)
    pdf = pd.astype(_f32)
    u = (lax.broadcasted_iota(_i32, (128, 128), 0)
         <= lax.broadcasted_iota(_i32, (128, 128), 1)).astype(_f32)
    incl_s = jnp.dot(psf, u, preferred_element_type=_f32)
    incl_d = jnp.dot(pdf, u, preferred_element_type=_f32)
    sl = (lax.broadcasted_iota(_i32, (nrow, nrow), 0)
          > lax.broadcasted_iota(_i32, (nrow, nrow), 1)).astype(_f32)
    off_s = jnp.dot(sl, incl_s[:, 127:128], preferred_element_type=_f32)
    off_d = jnp.dot(sl, incl_d[:, 127:128], preferred_element_type=_f32)
    rs = off_s + incl_s - psf
    rd = off_d + incl_d - pdf
    n_src = jnp.sum(psf)
    jf = jnp.where(pd & (rd < n_src), rd, float(JSENT)).astype(_i32)
    pos = jnp.where(ps, rs, float(PTRASH)).astype(_i32)
    jf_ref[...] = jf[: NP // 128]
    pos_ref[...] = pos[: NP // 128]
    pd_ref[...] = pd[: NP // 128].astype(_i32)
    rowmask = (lax.broadcasted_iota(_i32, (NT, D), 0) < N).astype(_f32)
    tt = jnp.concatenate([t2_ref[0], t2_ref[1]], axis=1)
    text_ref[pl.ds(0, NT), :] = tt * rowmask
    text_ref[pl.ds(NT, NR - NT), :] = jnp.zeros((NR - NT, D), _f32)


_tc_rank = pl.pallas_call(
    _tc_rank_body,
    out_shape=[
        jax.ShapeDtypeStruct((NR, D), _f32),
        jax.ShapeDtypeStruct((NP // 128, 128), _i32),
        jax.ShapeDtypeStruct((NP // 128, 128), _i32),
        jax.ShapeDtypeStruct((NP // 128, 128), _i32),
    ],
)


@functools.partial(
    pl.kernel,
    out_type=jax.ShapeDtypeStruct((NP, D), _f32),
    mesh=_mesh,
    compiler_params=pltpu.CompilerParams(needs_layout_passes=False, use_tc_tiling_on_sc=False),
    scratch_types=[
        pltpu.VMEM((NI,), _i32),
        pltpu.VMEM((512,), _i32),
        pltpu.VMEM((320,), _i32),
        pltpu.VMEM((320,), _i32),
        pltpu.VMEM((32, D), _f32),
    ],
)
def _sc_gather_ranks(text_hbm, pos_hbm, jf_hbm, zi_hbm,
                     g_out,
                     inv, posv, jfv, idx2, grows):
    c = lax.axis_index("c")
    s = lax.axis_index("s")
    wid = c * NS + s
    iota = lax.iota(_i32, 16)

    pltpu.sync_copy(zi_hbm, inv)

    for k in range(NP // 512):
        pltpu.sync_copy(pos_hbm.at[pl.ds(k * 512, 512)], posv)

        def inv_body(t, carry, k=k):
            pv = posv[pl.ds(t * 16, 16)]
            plsc.store_scatter(inv, [pv], iota + (k * 512 + t * 16))
            return carry
        lax.fori_loop(0, 32, inv_body, 0)
    plsc.store_scatter(inv, [iota + JSENT], jnp.full((16,), SENT, _i32))

    base = wid * 320
    pltpu.sync_copy(jf_hbm.at[pl.ds(base, 320)], jfv)
    for k in range(20):
        jv = jfv[pl.ds(k * 16, 16)]
        uv = plsc.load_gather(inv, [jv])
        idx2[pl.ds(k * 16, 16)] = uv
    for t in range(10):
        pltpu.sync_copy(text_hbm.at[idx2.at[pl.ds(t * 32, 32)]], grows)
        pltpu.sync_copy(grows, g_out.at[pl.ds(base + t * 32, 32)])


def _dense_update(g_ref, h_ref, x_ref, pdc_ref, wl_ref, w0_ref, w1_ref,
                  bl_ref, b0_ref, b1_ref, lng_ref, lnb_ref):
    dims = (((1,), (1,)), ((), ()))
    hd = lax.dot_general(g_ref[...], wl_ref[...], dims,
                         preferred_element_type=_f32)
    hd = hd + lax.dot_general(h_ref[...], w0_ref[...], dims,
                              preferred_element_type=_f32)
    hd = hd + lax.dot_general(x_ref[...], w1_ref[...], dims,
                              preferred_element_type=_f32)
    hd = hd + (bl_ref[...] + b0_ref[...] + b1_ref[...])
    hd = jnp.maximum(hd, 0.0)
    mu = jnp.mean(hd, axis=-1, keepdims=True)
    var = jnp.mean((hd - mu) * (hd - mu), axis=-1, keepdims=True)
    hd = (hd - mu) * lax.rsqrt(var + 1e-5) * lng_ref[...] + lnb_ref[...]
    h = h_ref[...]
    return h + pdc_ref[...] * (hd - h)


def _tc_update_body(g_ref, h_ref, x_ref, pdc_ref, wl_ref, w0_ref, w1_ref,
                    bl_ref, b0_ref, b1_ref, lng_ref, lnb_ref, hext_ref):
    hn = _dense_update(g_ref, h_ref, x_ref, pdc_ref, wl_ref, w0_ref, w1_ref,
                       bl_ref, b0_ref, b1_ref, lng_ref, lnb_ref)
    hext_ref[pl.ds(0, N), :] = hn
    hext_ref[pl.ds(N, NR - N), :] = jnp.zeros((NR - N, D), _f32)


def _tc_final_body(h_ref, ow_ref, ob_ref, out_ref):
    dims = (((1,), (1,)), ((), ()))
    out_ref[...] = lax.dot_general(h_ref[...], ow_ref[...], dims,
                                   preferred_element_type=_f32) + ob_ref[...]


_tc_update = pl.pallas_call(
    _tc_update_body, out_shape=jax.ShapeDtypeStruct((NR, D), _f32))
_tc_final = pl.pallas_call(
    _tc_final_body, out_shape=jax.ShapeDtypeStruct((N, 128), _f32))


def _prep_edges(ei):
    pad = NS * EPS - E
    src = jnp.concatenate([ei[0], jnp.full((pad,), SENT, _i32)])
    dst = jnp.concatenate([ei[1], jnp.full((pad,), SENT, _i32)])
    return jnp.concatenate([src.reshape(NS, CH, CW), dst.reshape(NS, CH, CW)],
                           axis=2)


@jax.jit
def kernel(x, edge_index0, edge_index1, wl0_W, wl0_b, w00_W, w00_b, w10_W,
           w10_b, ln0_g, ln0_b, wl1_W, wl1_b, w01_W, w01_b, w11_W, w11_b,
           ln1_g, ln1_b, out_W, out_b):
    ones8 = jnp.ones((CW, 8), _i32)
    zt = jnp.zeros((NT // NS, DH), _f32)
    zp = jnp.zeros((NB, 8), _i32)
    zi = jnp.zeros((NI,), _i32)
    row = lambda b: b.reshape(1, D)

    convs = [
        (_prep_edges(edge_index1), wl1_W, row(wl1_b), w01_W, row(w01_b),
         w11_W, row(w11_b), row(ln1_g), row(ln1_b)),
        (_prep_edges(edge_index0), wl0_W, row(wl0_b), w00_W, row(w00_b),
         w10_W, row(w10_b), row(ln0_g), row(ln0_b)),
    ]

    def conv_step(h_ext, xv):
        (sd2d, wl, bl, w0, b0, w1, b1, lng, lnb) = xv
        hlo = h_ext[:, :DH]
        hhi = h_ext[:, DH:]
        t2, bs, bd = _sc_edge_agg(hlo, hhi, sd2d, ones8, zt, zp)
        bs1 = bs[:, :, 0].reshape(NS, NB // 128, 128)
        bd1 = bd[:, :, 0].reshape(NS, NB // 128, 128)
        text, jf, pos, pdm = _tc_rank(t2, bs1, bd1)
        g = _sc_gather_ranks(text, pos.reshape(NP), jf.reshape(NP), zi)
        pdc = pdm.reshape(NP)[:N, None].astype(_f32)
        h_new = _tc_update(g[:N], h_ext[:N], x, pdc, wl, w0, w1,
                           bl, b0, b1, lng, lnb)
        return h_new

    h_ext = jnp.pad(x, ((0, NR - N), (0, 0)))
    for xv in convs:
        h_ext = conv_step(h_ext, xv)
    return _tc_final(h_ext[:N], out_W, row(out_b))


# speedup vs baseline: 1.0366x; 1.0008x over previous
import functools
import jax
import jax.numpy as jnp
from jax import lax
from jax.experimental import pallas as pl
from jax.experimental.pallas import tpu as pltpu
from jax.experimental.pallas import tpu_sc as plsc

N = 10000
D = 128
E = 320000

NC = 2
NS = 16
NW = NC * NS

NR = 10368
NP = 10240
NB = 10368
NT = 10112
NI = 10048
DH = 64
EPS = 20224
CH = 316
CW = 64
SENT = 10000
JSENT = 10016
PTRASH = 10032

_mesh = plsc.VectorSubcoreMesh(core_axis_name="c", subcore_axis_name="s")
_f32 = jnp.float32
_i32 = jnp.int32


@functools.partial(
    pl.kernel,
    out_type=[
        jax.ShapeDtypeStruct((NC, NT, DH), _f32),
        jax.ShapeDtypeStruct((NS, NB, 8), _i32),
        jax.ShapeDtypeStruct((NS, NB, 8), _i32),
    ],
    mesh=_mesh,
    compiler_params=pltpu.CompilerParams(needs_layout_passes=False, use_tc_tiling_on_sc=False),
    scratch_types=[
        pltpu.VMEM((1, 2 * CW), _i32),
        pltpu.VMEM((1, 2 * CW), _i32),
        pltpu.VMEM((CW, DH), _f32),
        pltpu.VMEM((CW, DH), _f32),
        pltpu.VMEM((CW, 8), _i32),
        pltpu.VMEM_SHARED((NT, DH), _f32),
        pltpu.SemaphoreType.DMA,
        pltpu.SemaphoreType.DMA,
        pltpu.SemaphoreType.DMA,
        pltpu.SemaphoreType.DMA,
        pltpu.SemaphoreType.DMA,
        pltpu.SemaphoreType.DMA,
        pltpu.SemaphoreType.DMA,
        pltpu.SemaphoreType.DMA,
    ],
)
def _sc_edge_agg(hlo_hbm, hhi_hbm, sd_hbm, ones_hbm, zt_hbm,
                 zp_hbm,
                 t_out, ps_out, pd_out,
                 ib0, ib1, rows0, rows1, ones8, t_sp,
                 semi0, semi1, semg0, semg1,
                 semsc0, semsc1, semp0, semp1):
    c = lax.axis_index("c")
    s = lax.axis_index("s")
    rpt = NT // NS

    pltpu.sync_copy(ones_hbm, ones8)
    pltpu.sync_copy(zt_hbm, t_sp.at[pl.ds(s * rpt, rpt)])

    @pl.when(c == 0)
    def _():
        pltpu.sync_copy(zp_hbm, ps_out.at[s])
        pltpu.sync_copy(zp_hbm, pd_out.at[s])

    plsc.subcore_barrier()

    ib = (ib0, ib1)
    rows = (rows0, rows1)
    semi = (semi0, semi1)
    semg = (semg0, semg1)
    semsc = (semsc0, semsc1)
    semp = (semp0, semp1)

    def fire_idx(q, p):
        pltpu.async_copy(sd_hbm.at[s, pl.ds(q, 1)], ib[p], semi[p])

    def wait_idx(p):
        pltpu.make_async_copy(sd_hbm.at[s, pl.ds(0, 1)], ib[p],
                              semi[p]).wait()

    def wait_rows(sem, p):
        pltpu.make_async_copy(hlo_hbm.at[pl.ds(0, CW // 2)],
                              rows[p].at[pl.ds(0, CW // 2)], sem).wait()
        pltpu.make_async_copy(hlo_hbm.at[pl.ds(0, CW // 2)],
                              rows[p].at[pl.ds(0, CW // 2)], sem).wait()

    def wait_pres(p):
        pltpu.make_async_copy(zp_hbm.at[pl.ds(0, CW)], ones8, semp[p]).wait()
        pltpu.make_async_copy(zp_hbm.at[pl.ds(0, CW)], ones8, semp[p]).wait()

    fire_idx(0, 0)
    fire_idx(1, 1)

    def halfstep(i, p):
        wait_idx(p)

        isrc = ib[p].at[0, pl.ds(0, CW)]
        idst = ib[p].at[0, pl.ds(CW, CW)]
        hw = CW // 2
        idst_a = ib[p].at[0, pl.ds(CW, hw)]
        idst_b = ib[p].at[0, pl.ds(CW + hw, hw)]
        isrc_a = ib[p].at[0, pl.ds(0, hw)]
        isrc_b = ib[p].at[0, pl.ds(hw, hw)]
        rlo = rows[p].at[pl.ds(0, hw)]
        rhi = rows[p].at[pl.ds(hw, hw)]

        @pl.when(c == 0)
        def _():
            pltpu.async_copy(hlo_hbm.at[idst_a], rlo, semg[p])
            pltpu.async_copy(hlo_hbm.at[idst_b], rhi, semg[p])

        @pl.when(c == 1)
        def _():
            pltpu.async_copy(hhi_hbm.at[idst_a], rlo, semg[p])
            pltpu.async_copy(hhi_hbm.at[idst_b], rhi, semg[p])

        wait_rows(semg[p], p)
        pltpu.async_copy(rlo, t_sp.at[isrc_a], semsc[p], add=True)
        pltpu.async_copy(rhi, t_sp.at[isrc_b], semsc[p], add=True)

        @pl.when(c == 0)
        def _():
            pltpu.async_copy(ones8, ps_out.at[s].at[isrc], semp[p])
            pltpu.async_copy(ones8, pd_out.at[s].at[idst], semp[p])

    def refill(i, p):
        wait_rows(semsc[p], p)

        @pl.when(c == 0)
        def _():
            wait_pres(p)

        @pl.when(i < CH // 2 - 1)
        def _():
            fire_idx(2 * i + 2 + p, p)

    def body(i, carry):
        halfstep(i, 0)
        halfstep(i, 1)
        refill(i, 0)
        refill(i, 1)
        return carry

    lax.fori_loop(0, CH // 2, body, 0)

    plsc.subcore_barrier()

    pltpu.sync_copy(t_sp.at[pl.ds(s * rpt, rpt)],
                    t_out.at[c, pl.ds(s * rpt, rpt)])


def _tc_rank_body(t2_ref, bs_ref, bd_ref, text_ref, jf_ref, pos_ref, pd_ref):
    cs_t = jnp.sum(bs_ref[...], axis=0)
    cd_t = jnp.sum(bd_ref[...], axis=0)
    nrow = NB // 128
    r = lax.broadcasted_iota(_i32, (nrow, 128), 0)
    cl = lax.broadcasted_iota(_i32, (nrow, 128), 1)
    v = r * 128 + cl
    valid = v < N
    ps = (cs_t > 0) & valid
    pd = (cd_t > 0) & valid
    psf = ps.astype(_f32)
    pdf = pd.astype(_f32)
    u = (lax.broadcasted_iota(_i32, (128, 128), 0)
         <= lax.broadcasted_iota(_i32, (128, 128), 1)).astype(_f32)
    incl_s = jnp.dot(psf, u, preferred_element_type=_f32)
    incl_d = jnp.dot(pdf, u, preferred_element_type=_f32)
    sl = (lax.broadcasted_iota(_i32, (nrow, nrow), 0)
          > lax.broadcasted_iota(_i32, (nrow, nrow), 1)).astype(_f32)
    off_s = jnp.dot(sl, incl_s[:, 127:128], preferred_element_type=_f32)
    off_d = jnp.dot(sl, incl_d[:, 127:128], preferred_element_type=_f32)
    rs = off_s + incl_s - psf
    rd = off_d + incl_d - pdf
    n_src = jnp.sum(psf)
    jf = jnp.where(pd & (rd < n_src), rd, float(JSENT)).astype(_i32)
    pos = jnp.where(ps, rs, float(PTRASH)).astype(_i32)
    jf_ref[...] = jf[: NP // 128]
    pos_ref[...] = pos[: NP // 128]
    pd_ref[...] = pd[: NP // 128].astype(_i32)
    rowmask = (lax.broadcasted_iota(_i32, (NT, D), 0) < N).astype(_f32)
    tt = jnp.concatenate([t2_ref[0], t2_ref[1]], axis=1)
    text_ref[pl.ds(0, NT), :] = tt * rowmask
    text_ref[pl.ds(NT, NR - NT), :] = jnp.zeros((NR - NT, D), _f32)


_tc_rank = pl.pallas_call(
    _tc_rank_body,
    out_shape=[
        jax.ShapeDtypeStruct((NR, D), _f32),
        jax.ShapeDtypeStruct((NP // 128, 128), _i32),
        jax.ShapeDtypeStruct((NP // 128, 128), _i32),
        jax.ShapeDtypeStruct((NP // 128, 128), _i32),
    ],
)


@functools.partial(
    pl.kernel,
    out_type=jax.ShapeDtypeStruct((NP, D), _f32),
    mesh=_mesh,
    compiler_params=pltpu.CompilerParams(needs_layout_passes=False, use_tc_tiling_on_sc=False),
    scratch_types=[
        pltpu.VMEM((NI,), _i32),
        pltpu.VMEM((512,), _i32),
        pltpu.VMEM((320,), _i32),
        pltpu.VMEM((320,), _i32),
        pltpu.VMEM((32, D), _f32),
    ],
)
def _sc_gather_ranks(text_hbm, pos_hbm, jf_hbm, zi_hbm,
                     g_out,
                     inv, posv, jfv, idx2, grows):
    c = lax.axis_index("c")
    s = lax.axis_index("s")
    wid = c * NS + s
    iota = lax.iota(_i32, 16)

    pltpu.sync_copy(zi_hbm, inv)

    for k in range(NP // 512):
        pltpu.sync_copy(pos_hbm.at[pl.ds(k * 512, 512)], posv)

        def inv_body(t, carry, k=k):
            pv = posv[pl.ds(t * 16, 16)]
            plsc.store_scatter(inv, [pv], iota + (k * 512 + t * 16))
            return carry
        lax.fori_loop(0, 32, inv_body, 0)
    plsc.store_scatter(inv, [iota + JSENT], jnp.full((16,), SENT, _i32))

    base = wid * 320
    pltpu.sync_copy(jf_hbm.at[pl.ds(base, 320)], jfv)
    for k in range(20):
        jv = jfv[pl.ds(k * 16, 16)]
        uv = plsc.load_gather(inv, [jv])
        idx2[pl.ds(k * 16, 16)] = uv
    for t in range(10):
        pltpu.sync_copy(text_hbm.at[idx2.at[pl.ds(t * 32, 32)]], grows)
        pltpu.sync_copy(grows, g_out.at[pl.ds(base + t * 32, 32)])


def _dense_update(g_ref, h_ref, x_ref, pdc_ref, wl_ref, w0_ref, w1_ref,
                  bl_ref, b0_ref, b1_ref, lng_ref, lnb_ref):
    dims = (((1,), (1,)), ((), ()))
    hd = lax.dot_general(g_ref[...], wl_ref[...], dims,
                         preferred_element_type=_f32)
    hd = hd + lax.dot_general(h_ref[...], w0_ref[...], dims,
                              preferred_element_type=_f32)
    hd = hd + lax.dot_general(x_ref[...], w1_ref[...], dims,
                              preferred_element_type=_f32)
    hd = hd + (bl_ref[...] + b0_ref[...] + b1_ref[...])
    hd = jnp.maximum(hd, 0.0)
    mu = jnp.mean(hd, axis=-1, keepdims=True)
    var = jnp.mean((hd - mu) * (hd - mu), axis=-1, keepdims=True)
    hd = (hd - mu) * lax.rsqrt(var + 1e-5) * lng_ref[...] + lnb_ref[...]
    h = h_ref[...]
    return h + pdc_ref[...] * (hd - h)


def _tc_update_body(g_ref, h_ref, x_ref, pdc_ref, wl_ref, w0_ref, w1_ref,
                    bl_ref, b0_ref, b1_ref, lng_ref, lnb_ref, hext_ref):
    hn = _dense_update(g_ref, h_ref, x_ref, pdc_ref, wl_ref, w0_ref, w1_ref,
                       bl_ref, b0_ref, b1_ref, lng_ref, lnb_ref)
    hext_ref[pl.ds(0, N), :] = hn
    hext_ref[pl.ds(N, NR - N), :] = jnp.zeros((NR - N, D), _f32)


def _tc_final_body(h_ref, ow_ref, ob_ref, out_ref):
    dims = (((1,), (1,)), ((), ()))
    out_ref[...] = lax.dot_general(h_ref[...], ow_ref[...], dims,
                                   preferred_element_type=_f32) + ob_ref[...]


_tc_update = pl.pallas_call(
    _tc_update_body, out_shape=jax.ShapeDtypeStruct((NR, D), _f32))
_tc_final = pl.pallas_call(
    _tc_final_body, out_shape=jax.ShapeDtypeStruct((N, 128), _f32))


@functools.partial(
    pl.kernel,
    out_type=jax.ShapeDtypeStruct((NS, 16), _i32),
    mesh=_mesh,
    compiler_params=pltpu.CompilerParams(needs_layout_passes=False, use_tc_tiling_on_sc=False),
    scratch_types=[pltpu.VMEM((16,), _i32)],
)
def _sc_warmup(src_hbm, o_hbm, buf):
    c = lax.axis_index("c")
    s = lax.axis_index("s")

    @pl.when(c == 0)
    def _():
        pltpu.sync_copy(src_hbm.at[s, 0, pl.ds(0, 16)], buf)
        pltpu.sync_copy(buf, o_hbm.at[s])


def _prep_edges(ei):
    pad = NS * EPS - E
    src = jnp.concatenate([ei[0], jnp.full((pad,), SENT, _i32)])
    dst = jnp.concatenate([ei[1], jnp.full((pad,), SENT, _i32)])
    return jnp.concatenate([src.reshape(NS, CH, CW), dst.reshape(NS, CH, CW)],
                           axis=2)


@jax.jit
def kernel(x, edge_index0, edge_index1, wl0_W, wl0_b, w00_W, w00_b, w10_W,
           w10_b, ln0_g, ln0_b, wl1_W, wl1_b, w01_W, w01_b, w11_W, w11_b,
           ln1_g, ln1_b, out_W, out_b):
    ones8 = jnp.ones((CW, 8), _i32)
    zt = jnp.zeros((NT // NS, DH), _f32)
    zp = jnp.zeros((NB, 8), _i32)
    zi = jnp.zeros((NI,), _i32)
    row = lambda b: b.reshape(1, D)

    convs = [
        (_prep_edges(edge_index1), wl1_W, row(wl1_b), w01_W, row(w01_b),
         w11_W, row(w11_b), row(ln1_g), row(ln1_b)),
        (_prep_edges(edge_index0), wl0_W, row(wl0_b), w00_W, row(w00_b),
         w10_W, row(w10_b), row(ln0_g), row(ln0_b)),
    ]

    def conv_step(h_ext, xv):
        (sd2d, wl, bl, w0, b0, w1, b1, lng, lnb) = xv
        hlo = h_ext[:, :DH]
        hhi = h_ext[:, DH:]
        t2, bs, bd = _sc_edge_agg(hlo, hhi, sd2d, ones8, zt, zp)
        bs1 = bs[:, :, 0].reshape(NS, NB // 128, 128)
        bd1 = bd[:, :, 0].reshape(NS, NB // 128, 128)
        text, jf, pos, pdm = _tc_rank(t2, bs1, bd1)
        g = _sc_gather_ranks(text, pos.reshape(NP), jf.reshape(NP), zi)
        pdc = pdm.reshape(NP)[:N, None].astype(_f32)
        h_new = _tc_update(g[:N], h_ext[:N], x, pdc, wl, w0, w1,
                           bl, b0, b1, lng, lnb)
        return h_new

    h_ext = jnp.pad(x, ((0, NR - N), (0, 0)))
    wz = _sc_warmup(convs[0][0])
    h_ext = h_ext + (wz[0, 0] - wz[0, 0]).astype(_f32)
    for xv in convs:
        h_ext = conv_step(h_ext, xv)
    return _tc_final(h_ext[:N], out_W, row(out_b))
